# baseline (speedup 1.0000x reference)
import functools

import jax
import jax.numpy as jnp
from jax.experimental import pallas as pl
from jax.experimental.pallas import tpu as pltpu
from jax.experimental.pallas import tpu_sc as plsc

_BLK = 1024
_CH = 8
_GW = 512
_EP = 128
_NPAD = 4096


def _sc_gather(emb_p, idx):
    n = idx.shape[0]
    ep = emb_p.shape[1]
    NC, NS = 2, 16
    NW = NC * NS
    per_w = n // NW
    SUP = 12800
    n_sup = per_w // SUP
    CHUNK = _GW
    n_ch = SUP // CHUNK
    mesh = plsc.VectorSubcoreMesh(core_axis_name="core", subcore_axis_name="subcore")

    @functools.partial(
        pl.kernel,
        out_type=jax.ShapeDtypeStruct((n, ep), emb_p.dtype),
        mesh=mesh,
        scratch_types=[
            pltpu.VMEM((SUP,), jnp.int32),
            pltpu.VMEM((CHUNK, ep), emb_p.dtype),
        ],
    )
    def k(emb_hbm, idx_hbm, out_hbm, idx_v, rows_v):
        core = jax.lax.axis_index("core")
        sub = jax.lax.axis_index("subcore")
        base = (sub * NC + core) * per_w

        @pl.loop(0, n_sup)
        def _sup(s):
            pltpu.sync_copy(idx_hbm.at[pl.ds(base + s * SUP, SUP)], idx_v)

            @pl.loop(0, n_ch)
            def _ch(c):
                pltpu.sync_copy(emb_hbm.at[idx_v.at[pl.ds(c * CHUNK, CHUNK)]],
                                rows_v)
                pltpu.sync_copy(
                    rows_v,
                    out_hbm.at[pl.ds(base + s * SUP + c * CHUNK, CHUNK)])

    return k(emb_p, idx)


def _gru_pallas(xe2, ls_col, lastchunk, wihT, whhT, b2):
    Bs = xe2.shape[0]
    L = xe2.shape[1] // _EP
    H = whhT.shape[0]
    G = whhT.shape[1]
    R = Bs // _BLK
    NT = L // _CH

    def body(s_ref, xe_ref, len_ref, wih_ref, whh_ref, b_ref, o_ref, h_ref):
        c = pl.program_id(1)

        @pl.when(c == 0)
        def _init():
            h_ref[...] = jnp.zeros_like(h_ref)

        @pl.when(c <= s_ref[pl.program_id(0)])
        def _compute():
            h = h_ref[...]
            lens = len_ref[...]
            bih = b_ref[0:1, :]
            bhh = b_ref[1:2, :]
            for tt in range(_CH):
                t = c * _CH + tt
                xe_t = xe_ref[:, tt * _EP:(tt + 1) * _EP]
                gi = jnp.dot(xe_t, wih_ref[...],
                             preferred_element_type=jnp.float32) + bih
                gh = jnp.dot(h, whh_ref[...],
                             preferred_element_type=jnp.float32) + bhh
                gsum = gi + gh
                rz = jax.nn.sigmoid(gsum[:, : 2 * H])
                rr = rz[:, :H]
                zz = rz[:, H:]
                n = jnp.tanh(gi[:, 2 * H:] + rr * gh[:, 2 * H:])
                h_new = (1.0 - zz) * n + zz * h
                h = jnp.where(lens > t, h_new, h)
            h_ref[...] = h

        o_ref[...] = h_ref[...]

    return pl.pallas_call(
        body,
        grid_spec=pltpu.PrefetchScalarGridSpec(
            num_scalar_prefetch=1,
            grid=(R, NT),
            in_specs=[
                pl.BlockSpec((_BLK, _CH * _EP),
                             lambda r, c, s: (r, jnp.minimum(c, s[r]))),
                pl.BlockSpec((_BLK, 1), lambda r, c, s: (r, 0)),
                pl.BlockSpec((_EP, G), lambda r, c, s: (0, 0)),
                pl.BlockSpec((H, G), lambda r, c, s: (0, 0)),
                pl.BlockSpec((8, G), lambda r, c, s: (0, 0)),
            ],
            out_specs=pl.BlockSpec((_BLK, H), lambda r, c, s: (r, 0)),
            scratch_shapes=[pltpu.VMEM((_BLK, H), jnp.float32)],
        ),
        out_shape=jax.ShapeDtypeStruct((Bs, H), jnp.float32),
        compiler_params=pltpu.CompilerParams(
            dimension_semantics=("parallel", "arbitrary")),
    )(lastchunk, xe2, ls_col, wihT, whhT, b2)


def kernel(x, emb, W_ih, W_hh, b_ih, b_hh):
    B, L = x.shape
    V, E = emb.shape
    H = W_hh.shape[1]

    l = jnp.sum(x != 0, axis=1).astype(jnp.int32)
    perm = jnp.argsort(-l)
    R = B // _BLK
    order = jnp.concatenate([jnp.arange(0, R, 2), jnp.arange(1, R, 2)])
    perm = perm.reshape(R, _BLK)[order].reshape(B)
    ls = l[perm]
    xs = x[perm]

    emb_p = jnp.pad(emb, ((0, _NPAD), (0, _EP - E)))
    flat = xs.reshape(B * L)
    spread = (jnp.arange(B * L, dtype=jnp.int32) % _NPAD) + V
    xe = _sc_gather(emb_p, jnp.where(flat == 0, spread, flat))
    xe2 = xe.reshape(B, L * _EP)

    block_max = ls.reshape(R, _BLK).max(axis=1)
    lastchunk = (jnp.maximum((block_max + _CH - 1) // _CH, 1) - 1).astype(jnp.int32)
    ls_col = ls[:, None]
    b2 = jnp.zeros((8, 3 * H), jnp.float32).at[0].set(b_ih).at[1].set(b_hh)

    wihT_p = jnp.pad(W_ih.T, ((0, _EP - E), (0, 0)))
    h = _gru_pallas(xe2, ls_col, lastchunk, wihT_p, W_hh.T, b2)

    nonempty = l != 0
    dest = jnp.where(nonempty, jnp.cumsum(nonempty.astype(jnp.int32)) - 1, B)
    retval = jnp.zeros((B, H), jnp.float32).at[dest[perm]].set(h, mode="drop")
    return retval


# speedup vs baseline: 1.3737x; 1.3737x over previous
import functools

import jax
import jax.numpy as jnp
from jax.experimental import pallas as pl
from jax.experimental.pallas import tpu as pltpu
from jax.experimental.pallas import tpu_sc as plsc

_BLK = 1024
_CH = 8
_GW = 512
_EP = 128
_NPAD = 4096


def _sc_gather(emb_p, idx):
    n = idx.shape[0]
    ep = emb_p.shape[1]
    NC, NS = 2, 16
    NW = NC * NS
    per_w = n // NW
    SUP = 12800
    n_sup = per_w // SUP
    CHUNK = _GW
    n_ch = SUP // CHUNK
    mesh = plsc.VectorSubcoreMesh(core_axis_name="core", subcore_axis_name="subcore")

    @functools.partial(
        pl.kernel,
        out_type=jax.ShapeDtypeStruct((n, ep), emb_p.dtype),
        mesh=mesh,
        scratch_types=[
            pltpu.VMEM((SUP,), jnp.int32),
            pltpu.VMEM((CHUNK, ep), emb_p.dtype),
        ],
    )
    def k(emb_hbm, idx_hbm, out_hbm, idx_v, rows_v):
        core = jax.lax.axis_index("core")
        sub = jax.lax.axis_index("subcore")
        base = (sub * NC + core) * per_w

        @pl.loop(0, n_sup)
        def _sup(s):
            pltpu.sync_copy(idx_hbm.at[pl.ds(base + s * SUP, SUP)], idx_v)

            @pl.loop(0, n_ch)
            def _ch(c):
                pltpu.sync_copy(emb_hbm.at[idx_v.at[pl.ds(c * CHUNK, CHUNK)]],
                                rows_v)
                pltpu.sync_copy(
                    rows_v,
                    out_hbm.at[pl.ds(base + s * SUP + c * CHUNK, CHUNK)])

    return k(emb_p, idx)


def _gru_pallas(xe2, ls_col, lastchunk, wihT, whhT, b2):
    Bs, L = xe2.shape[0], xe2.shape[1]
    H = whhT.shape[0]
    G = whhT.shape[1]
    R = Bs // _BLK
    NT = L // _CH

    def body(s_ref, xe_ref, len_ref, wih_ref, whh_ref, b_ref, o_ref, h_ref):
        c = pl.program_id(1)

        @pl.when(c == 0)
        def _init():
            h_ref[...] = jnp.zeros_like(h_ref)

        @pl.when(c <= s_ref[pl.program_id(0)])
        def _compute():
            h = h_ref[...]
            lens = len_ref[...]
            bih = b_ref[0:1, :]
            bhh = b_ref[1:2, :]
            for tt in range(_CH):
                t = c * _CH + tt
                xe_t = xe_ref[:, tt, :]
                gi = jnp.dot(xe_t, wih_ref[...],
                             preferred_element_type=jnp.float32) + bih
                gh = jnp.dot(h, whh_ref[...],
                             preferred_element_type=jnp.float32) + bhh
                gsum = gi + gh
                rz = jax.nn.sigmoid(gsum[:, : 2 * H])
                rr = rz[:, :H]
                zz = rz[:, H:]
                n = jnp.tanh(gi[:, 2 * H:] + rr * gh[:, 2 * H:])
                h_new = (1.0 - zz) * n + zz * h
                h = jnp.where(lens > t, h_new, h)
            h_ref[...] = h

        o_ref[...] = h_ref[...]

    return pl.pallas_call(
        body,
        grid_spec=pltpu.PrefetchScalarGridSpec(
            num_scalar_prefetch=1,
            grid=(R, NT),
            in_specs=[
                pl.BlockSpec((_BLK, _CH, _EP),
                             lambda r, c, s: (r, jnp.minimum(c, s[r]), 0)),
                pl.BlockSpec((_BLK, 1), lambda r, c, s: (r, 0)),
                pl.BlockSpec((_EP, G), lambda r, c, s: (0, 0)),
                pl.BlockSpec((H, G), lambda r, c, s: (0, 0)),
                pl.BlockSpec((8, G), lambda r, c, s: (0, 0)),
            ],
            out_specs=pl.BlockSpec((_BLK, H), lambda r, c, s: (r, 0)),
            scratch_shapes=[pltpu.VMEM((_BLK, H), jnp.float32)],
        ),
        out_shape=jax.ShapeDtypeStruct((Bs, H), jnp.float32),
        compiler_params=pltpu.CompilerParams(
            dimension_semantics=("parallel", "arbitrary")),
    )(lastchunk, xe2, ls_col, wihT, whhT, b2)


def kernel(x, emb, W_ih, W_hh, b_ih, b_hh):
    B, L = x.shape
    V, E = emb.shape
    H = W_hh.shape[1]

    l = jnp.sum(x != 0, axis=1).astype(jnp.int32)
    perm = jnp.argsort(-l)
    R = B // _BLK
    order = jnp.concatenate([jnp.arange(0, R, 2), jnp.arange(1, R, 2)])
    perm = perm.reshape(R, _BLK)[order].reshape(B)
    ls = l[perm]
    xs = x[perm]

    emb_p = jnp.pad(emb, ((0, _NPAD), (0, _EP - E)))
    flat = xs.reshape(B * L)
    spread = (jnp.arange(B * L, dtype=jnp.int32) % _NPAD) + V
    xe = _sc_gather(emb_p, jnp.where(flat == 0, spread, flat))
    xe2 = xe.reshape(B, L, _EP)

    block_max = ls.reshape(R, _BLK).max(axis=1)
    lastchunk = (jnp.maximum((block_max + _CH - 1) // _CH, 1) - 1).astype(jnp.int32)
    ls_col = ls[:, None]
    b2 = jnp.zeros((8, 3 * H), jnp.float32).at[0].set(b_ih).at[1].set(b_hh)

    wihT_p = jnp.pad(W_ih.T, ((0, _EP - E), (0, 0)))
    h = _gru_pallas(xe2, ls_col, lastchunk, wihT_p, W_hh.T, b2)

    nonempty = l != 0
    dest = jnp.where(nonempty, jnp.cumsum(nonempty.astype(jnp.int32)) - 1, B)
    retval = jnp.zeros((B, H), jnp.float32).at[dest[perm]].set(h, mode="drop")
    return retval


# speedup vs baseline: 1.4326x; 1.0429x over previous
import functools

import jax
import jax.numpy as jnp
from jax.experimental import pallas as pl
from jax.experimental.pallas import tpu as pltpu
from jax.experimental.pallas import tpu_sc as plsc

_BLK = 1024
_CH = 8
_GW = 512
_EP = 128
_NPAD = 4096
_NW = 32


def _sc_gather(emb_p, idx, b, l):
    n = idx.shape[0]
    ep = emb_p.shape[1]
    tpw = (l + _NW - 1) // _NW
    maxc = b // _GW
    mesh = plsc.VectorSubcoreMesh(core_axis_name="core", subcore_axis_name="subcore")

    @functools.partial(
        pl.kernel,
        out_type=jax.ShapeDtypeStruct((n, ep), emb_p.dtype),
        mesh=mesh,
        scratch_types=[
            pltpu.VMEM((b,), jnp.int32),
            pltpu.VMEM((_GW, ep), emb_p.dtype),
        ],
    )
    def k(emb_hbm, idx_hbm, out_hbm, idx_v, rows_v):
        core = jax.lax.axis_index("core")
        sub = jax.lax.axis_index("subcore")
        w = sub * 2 + core

        for j in range(tpw):
            t = w + j * _NW

            @pl.when(t < l)
            def _t():
                pltpu.sync_copy(idx_hbm.at[pl.ds(t * b, b)], idx_v)

                @pl.loop(0, maxc)
                def _c(c):
                    pltpu.sync_copy(
                        emb_hbm.at[idx_v.at[pl.ds(c * _GW, _GW)]],
                        rows_v)
                    pltpu.sync_copy(
                        rows_v,
                        out_hbm.at[pl.ds(t * b + c * _GW, _GW)])

    return k(emb_p, idx)


def _gru_pallas(xe3, ls_col, lastchunk, wihT, whhT, b2):
    L, Bs = xe3.shape[0], xe3.shape[1]
    H = whhT.shape[0]
    G = whhT.shape[1]
    R = Bs // _BLK
    NT = L // _CH

    def body(s_ref, xe_ref, len_ref, wih_ref, whh_ref, b_ref, o_ref, h_ref):
        c = pl.program_id(1)

        @pl.when(c == 0)
        def _init():
            h_ref[...] = jnp.zeros_like(h_ref)

        @pl.when(c <= s_ref[pl.program_id(0)])
        def _compute():
            h = h_ref[...]
            lens = len_ref[...]
            bih = b_ref[0:1, :]
            bhhn = b_ref[1:2, 2 * H:]
            for tt in range(_CH):
                t = c * _CH + tt
                xe_t = xe_ref[tt].astype(jnp.bfloat16)
                gi = jnp.dot(xe_t, wih_ref[...],
                             preferred_element_type=jnp.float32) + bih
                gh = jnp.dot(h.astype(jnp.bfloat16), whh_ref[...],
                             preferred_element_type=jnp.float32)
                rz = jax.nn.sigmoid(gi[:, :2 * H] + gh[:, :2 * H])
                rr = rz[:, :H]
                zz = rz[:, H:]
                n = jnp.tanh(gi[:, 2 * H:] + rr * (gh[:, 2 * H:] + bhhn))
                h_new = (1.0 - zz) * n + zz * h
                h = jnp.where(lens > t, h_new, h)
            h_ref[...] = h

        o_ref[...] = h_ref[...]

    return pl.pallas_call(
        body,
        grid_spec=pltpu.PrefetchScalarGridSpec(
            num_scalar_prefetch=1,
            grid=(R, NT),
            in_specs=[
                pl.BlockSpec((_CH, _BLK, _EP),
                             lambda r, c, s: (jnp.minimum(c, s[r]), r, 0)),
                pl.BlockSpec((_BLK, 1), lambda r, c, s: (r, 0)),
                pl.BlockSpec((_EP, G), lambda r, c, s: (0, 0)),
                pl.BlockSpec((H, G), lambda r, c, s: (0, 0)),
                pl.BlockSpec((8, G), lambda r, c, s: (0, 0)),
            ],
            out_specs=pl.BlockSpec((_BLK, H), lambda r, c, s: (r, 0)),
            scratch_shapes=[pltpu.VMEM((_BLK, H), jnp.float32)],
        ),
        out_shape=jax.ShapeDtypeStruct((Bs, H), jnp.float32),
        compiler_params=pltpu.CompilerParams(
            dimension_semantics=("arbitrary", "arbitrary")),
    )(lastchunk, xe3, ls_col, wihT, whhT, b2)


def kernel(x, emb, W_ih, W_hh, b_ih, b_hh):
    B, L = x.shape
    V, E = emb.shape
    H = W_hh.shape[1]

    l = jnp.sum(x != 0, axis=1).astype(jnp.int32)
    perm = jnp.argsort(-l)
    ls = l[perm]
    xs = x[perm]

    emb_p = jnp.pad(emb, ((0, _NPAD), (0, _EP - E)))
    flat = xs.T.reshape(L * B)
    spread = (jnp.arange(L * B, dtype=jnp.int32) % _NPAD) + V
    xe = _sc_gather(emb_p, jnp.where(flat == 0, spread, flat), B, L)
    xe3 = xe.reshape(L, B, _EP)

    R = B // _BLK
    block_max = ls.reshape(R, _BLK).max(axis=1)
    lastchunk = (jnp.maximum((block_max + _CH - 1) // _CH, 1) - 1).astype(jnp.int32)
    ls_col = ls[:, None]
    b2 = jnp.zeros((8, 3 * H), jnp.float32)
    b2 = b2.at[0].set(b_ih + jnp.concatenate([b_hh[:2 * H], jnp.zeros(H)]))
    b2 = b2.at[1, 2 * H:].set(b_hh[2 * H:])

    wihT_p = jnp.pad(W_ih.T, ((0, _EP - E), (0, 0))).astype(jnp.bfloat16)
    h = _gru_pallas(xe3, ls_col, lastchunk, wihT_p,
                    W_hh.T.astype(jnp.bfloat16), b2)

    nonempty = l != 0
    dest = jnp.where(nonempty, jnp.cumsum(nonempty.astype(jnp.int32)) - 1, B)
    retval = jnp.zeros((B, H), jnp.float32).at[dest[perm]].set(h, mode="drop")
    return retval


# speedup vs baseline: 2.0092x; 1.4025x over previous
import functools

import jax
import jax.numpy as jnp
from jax.experimental import pallas as pl
from jax.experimental.pallas import tpu as pltpu
from jax.experimental.pallas import tpu_sc as plsc

_BLK = 1024
_CH = 8
_GW = 512
_EP = 128
_NPAD = 4096
_NW = 32
_BS = 2048


def _sc_gather(emb_p, idx, b, l):
    n = idx.shape[0]
    ep = emb_p.shape[1]
    tpw = (l + _NW - 1) // _NW
    maxc = b // _GW
    mesh = plsc.VectorSubcoreMesh(core_axis_name="core", subcore_axis_name="subcore")

    @functools.partial(
        pl.kernel,
        out_type=jax.ShapeDtypeStruct((n, ep), emb_p.dtype),
        mesh=mesh,
        scratch_types=[
            pltpu.VMEM((b,), jnp.int32),
            pltpu.VMEM((_GW, ep), emb_p.dtype),
        ],
    )
    def k(emb_hbm, idx_hbm, out_hbm, idx_v, rows_v):
        core = jax.lax.axis_index("core")
        sub = jax.lax.axis_index("subcore")
        w = sub * 2 + core

        for j in range(tpw):
            t = w + j * _NW

            @pl.when(t < l)
            def _t():
                pltpu.sync_copy(idx_hbm.at[pl.ds(t * b, b)], idx_v)

                @pl.loop(0, maxc)
                def _c(c):
                    pltpu.sync_copy(
                        emb_hbm.at[idx_v.at[pl.ds(c * _GW, _GW)]],
                        rows_v)
                    pltpu.sync_copy(
                        rows_v,
                        out_hbm.at[pl.ds(t * b + c * _GW, _GW)])

    return k(emb_p, idx)


def _gru_pallas(xe3, ls_col, lastchunk, wihT, whhT, b2):
    L, Bs = xe3.shape[0], xe3.shape[1]
    H = whhT.shape[0]
    G = whhT.shape[1]
    R = Bs // _BLK
    NT = L // _CH

    def body(s_ref, xe_ref, len_ref, wih_ref, whh_ref, b_ref, o_ref, h_ref):
        c = pl.program_id(1)

        @pl.when(c == 0)
        def _init():
            h_ref[...] = jnp.zeros_like(h_ref)

        @pl.when(c <= s_ref[pl.program_id(0)])
        def _compute():
            h = h_ref[...]
            lens = len_ref[...]
            bih = b_ref[0:1, :]
            bhhn = b_ref[1:2, 2 * H:]
            for tt in range(_CH):
                t = c * _CH + tt
                xe_t = xe_ref[tt].astype(jnp.bfloat16)
                gi = jnp.dot(xe_t, wih_ref[...],
                             preferred_element_type=jnp.float32) + bih
                gh = jnp.dot(h.astype(jnp.bfloat16), whh_ref[...],
                             preferred_element_type=jnp.float32)
                rz = jax.nn.sigmoid(gi[:, :2 * H] + gh[:, :2 * H])
                rr = rz[:, :H]
                zz = rz[:, H:]
                n = jnp.tanh(gi[:, 2 * H:] + rr * (gh[:, 2 * H:] + bhhn))
                h_new = (1.0 - zz) * n + zz * h
                h = jnp.where(lens > t, h_new, h)
            h_ref[...] = h

        o_ref[...] = h_ref[...]

    return pl.pallas_call(
        body,
        grid_spec=pltpu.PrefetchScalarGridSpec(
            num_scalar_prefetch=1,
            grid=(R, NT),
            in_specs=[
                pl.BlockSpec((_CH, _BLK, _EP),
                             lambda r, c, s: (jnp.minimum(c, s[r]), r, 0)),
                pl.BlockSpec((_BLK, 1), lambda r, c, s: (r, 0)),
                pl.BlockSpec((_EP, G), lambda r, c, s: (0, 0)),
                pl.BlockSpec((H, G), lambda r, c, s: (0, 0)),
                pl.BlockSpec((8, G), lambda r, c, s: (0, 0)),
            ],
            out_specs=pl.BlockSpec((_BLK, H), lambda r, c, s: (r, 0)),
            scratch_shapes=[pltpu.VMEM((_BLK, H), jnp.float32)],
        ),
        out_shape=jax.ShapeDtypeStruct((Bs, H), jnp.float32),
        compiler_params=pltpu.CompilerParams(
            dimension_semantics=("arbitrary", "arbitrary")),
    )(lastchunk, xe3, ls_col, wihT, whhT, b2)


def kernel(x, emb, W_ih, W_hh, b_ih, b_hh):
    B, L = x.shape
    V, E = emb.shape
    H = W_hh.shape[1]

    l = jnp.sum(x != 0, axis=1).astype(jnp.int32)
    perm = jnp.argsort(-l)
    ls = l[perm]
    xs = x[perm]

    emb_p = jnp.pad(emb, ((0, _NPAD), (0, _EP - E)))
    spread2 = (jnp.arange(L * _BS, dtype=jnp.int32) % _NPAD) + V

    b2 = jnp.zeros((8, 3 * H), jnp.float32)
    b2 = b2.at[0].set(b_ih + jnp.concatenate([b_hh[:2 * H], jnp.zeros(H)]))
    b2 = b2.at[1, 2 * H:].set(b_hh[2 * H:])
    wihT_p = jnp.pad(W_ih.T, ((0, _EP - E), (0, 0))).astype(jnp.bfloat16)
    whhT_b = W_hh.T.astype(jnp.bfloat16)

    Rs = _BS // _BLK
    hs = []
    for s in range(B // _BS):
        xs_s = xs[s * _BS:(s + 1) * _BS]
        ls_s = ls[s * _BS:(s + 1) * _BS]
        flat = xs_s.T.reshape(L * _BS)
        xe = _sc_gather(emb_p, jnp.where(flat == 0, spread2, flat), _BS, L)
        xe3 = xe.reshape(L, _BS, _EP)
        block_max = ls_s.reshape(Rs, _BLK).max(axis=1)
        lastchunk = (jnp.maximum((block_max + _CH - 1) // _CH, 1) - 1
                     ).astype(jnp.int32)
        hs.append(_gru_pallas(xe3, ls_s[:, None], lastchunk, wihT_p,
                              whhT_b, b2))
    h = jnp.concatenate(hs, axis=0)

    nonempty = l != 0
    dest = jnp.where(nonempty, jnp.cumsum(nonempty.astype(jnp.int32)) - 1, B)
    retval = jnp.zeros((B, H), jnp.float32).at[dest[perm]].set(h, mode="drop")
    return retval


# speedup vs baseline: 2.0487x; 1.0197x over previous
import functools

import jax
import jax.numpy as jnp
from jax.experimental import pallas as pl
from jax.experimental.pallas import tpu as pltpu
from jax.experimental.pallas import tpu_sc as plsc

_BLK = 1024
_CH = 8
_GW = 256
_EP = 128
_NPAD = 4096
_NW = 32
_BS = 2048


def _sc_gather(emb_p, idx, b, l):
    n = idx.shape[0]
    ep = emb_p.shape[1]
    tpw = (l + _NW - 1) // _NW
    maxc = b // _GW
    mesh = plsc.VectorSubcoreMesh(core_axis_name="core", subcore_axis_name="subcore")

    @functools.partial(
        pl.kernel,
        out_type=jax.ShapeDtypeStruct((n, ep), emb_p.dtype),
        mesh=mesh,
        scratch_types=[
            pltpu.VMEM((b,), jnp.int32),
            pltpu.VMEM((2, _GW, ep), emb_p.dtype),
            pltpu.SemaphoreType.DMA,
            pltpu.SemaphoreType.DMA,
        ],
    )
    def k(emb_hbm, idx_hbm, out_hbm, idx_v, rows_v, sem0, sem1):
        core = jax.lax.axis_index("core")
        sub = jax.lax.axis_index("subcore")
        w = sub * 2 + core
        sems = (sem0, sem1)

        for j in range(tpw):
            t = w + j * _NW

            @pl.when(t < l)
            def _t():
                pltpu.sync_copy(idx_hbm.at[pl.ds(t * b, b)], idx_v)

                handles = [None, None]
                for c in range(maxc):
                    p = c & 1
                    if handles[p] is not None:
                        handles[p].wait()
                    pltpu.sync_copy(
                        emb_hbm.at[idx_v.at[pl.ds(c * _GW, _GW)]],
                        rows_v.at[p])
                    handles[p] = pltpu.async_copy(
                        rows_v.at[p],
                        out_hbm.at[pl.ds(t * b + c * _GW, _GW)],
                        sems[p])
                for p in range(2):
                    if handles[p] is not None:
                        handles[p].wait()

    return k(emb_p, idx)


def _gru_pallas(xe3, ls_col, lastchunk, wihT, whhT, b2):
    L, Bs = xe3.shape[0], xe3.shape[1]
    H = whhT.shape[0]
    G = whhT.shape[1]
    R = Bs // _BLK
    NT = L // _CH

    def body(s_ref, xe_ref, len_ref, wih_ref, whh_ref, b_ref, o_ref, h_ref):
        c = pl.program_id(1)

        @pl.when(c == 0)
        def _init():
            h_ref[...] = jnp.zeros_like(h_ref)

        @pl.when(c <= s_ref[pl.program_id(0)])
        def _compute():
            h = h_ref[...]
            lens = len_ref[...]
            bih = b_ref[0:1, :]
            bhhn = b_ref[1:2, 2 * H:]
            for tt in range(_CH):
                t = c * _CH + tt
                xe_t = xe_ref[tt].astype(jnp.bfloat16)
                gi = jnp.dot(xe_t, wih_ref[...],
                             preferred_element_type=jnp.float32) + bih
                gh = jnp.dot(h.astype(jnp.bfloat16), whh_ref[...],
                             preferred_element_type=jnp.float32)
                rz = jax.nn.sigmoid(gi[:, :2 * H] + gh[:, :2 * H])
                rr = rz[:, :H]
                zz = rz[:, H:]
                n = jnp.tanh(gi[:, 2 * H:] + rr * (gh[:, 2 * H:] + bhhn))
                h_new = (1.0 - zz) * n + zz * h
                h = jnp.where(lens > t, h_new, h)
            h_ref[...] = h

        o_ref[...] = h_ref[...]

    return pl.pallas_call(
        body,
        grid_spec=pltpu.PrefetchScalarGridSpec(
            num_scalar_prefetch=1,
            grid=(R, NT),
            in_specs=[
                pl.BlockSpec((_CH, _BLK, _EP),
                             lambda r, c, s: (jnp.minimum(c, s[r]), r, 0)),
                pl.BlockSpec((_BLK, 1), lambda r, c, s: (r, 0)),
                pl.BlockSpec((_EP, G), lambda r, c, s: (0, 0)),
                pl.BlockSpec((H, G), lambda r, c, s: (0, 0)),
                pl.BlockSpec((8, G), lambda r, c, s: (0, 0)),
            ],
            out_specs=pl.BlockSpec((_BLK, H), lambda r, c, s: (r, 0)),
            scratch_shapes=[pltpu.VMEM((_BLK, H), jnp.float32)],
        ),
        out_shape=jax.ShapeDtypeStruct((Bs, H), jnp.float32),
        compiler_params=pltpu.CompilerParams(
            dimension_semantics=("arbitrary", "arbitrary")),
    )(lastchunk, xe3, ls_col, wihT, whhT, b2)


def kernel(x, emb, W_ih, W_hh, b_ih, b_hh):
    B, L = x.shape
    V, E = emb.shape
    H = W_hh.shape[1]

    l = jnp.sum(x != 0, axis=1).astype(jnp.int32)
    perm = jnp.argsort(-l)
    ls = l[perm]
    xs = x[perm]

    emb_p = jnp.pad(emb, ((0, _NPAD), (0, _EP - E)))
    spread2 = (jnp.arange(L * _BS, dtype=jnp.int32) % _NPAD) + V

    b2 = jnp.zeros((8, 3 * H), jnp.float32)
    b2 = b2.at[0].set(b_ih + jnp.concatenate([b_hh[:2 * H], jnp.zeros(H)]))
    b2 = b2.at[1, 2 * H:].set(b_hh[2 * H:])
    wihT_p = jnp.pad(W_ih.T, ((0, _EP - E), (0, 0))).astype(jnp.bfloat16)
    whhT_b = W_hh.T.astype(jnp.bfloat16)

    Rs = _BS // _BLK
    hs = []
    for s in range(B // _BS):
        xs_s = xs[s * _BS:(s + 1) * _BS]
        ls_s = ls[s * _BS:(s + 1) * _BS]
        flat = xs_s.T.reshape(L * _BS)
        xe = _sc_gather(emb_p, jnp.where(flat == 0, spread2, flat), _BS, L)
        xe3 = xe.reshape(L, _BS, _EP)
        block_max = ls_s.reshape(Rs, _BLK).max(axis=1)
        lastchunk = (jnp.maximum((block_max + _CH - 1) // _CH, 1) - 1
                     ).astype(jnp.int32)
        hs.append(_gru_pallas(xe3, ls_s[:, None], lastchunk, wihT_p,
                              whhT_b, b2))
    h = jnp.concatenate(hs, axis=0)

    nonempty = l != 0
    dest = jnp.where(nonempty, jnp.cumsum(nonempty.astype(jnp.int32)) - 1, B)
    retval = jnp.zeros((B, H), jnp.float32).at[dest[perm]].set(h, mode="drop")
    return retval
